# Initial kernel scaffold; baseline (speedup 1.0000x reference)
import jax
import jax.numpy as jnp
from jax.experimental import pallas as pl


def kernel(TE, day_table, week_table):
    raise NotImplementedError("write your pallas kernel here")


# speedup vs baseline: 5.8436x; 5.8436x over previous
import functools

import jax
import jax.numpy as jnp
from jax import lax
from jax.experimental import pallas as pl
from jax.experimental.pallas import tpu as pltpu
from jax.experimental.pallas import tpu_sc as plsc

_NUM_WORKERS = 32
_CHUNK = 640
_GCH = 128
_NG = _CHUNK // _GCH


def _make_lookup(n_rows: int, d_out: int, n_tab: int):
  assert n_rows % (_NUM_WORKERS * _CHUNK) == 0
  rows_per_worker = n_rows // _NUM_WORKERS
  n_chunks = rows_per_worker // _CHUNK
  mesh = plsc.VectorSubcoreMesh(core_axis_name="c", subcore_axis_name="s")

  @functools.partial(
      pl.kernel,
      out_type=jax.ShapeDtypeStruct((n_rows, d_out), jnp.float32),
      mesh=mesh,
      scratch_types=[
          [pltpu.VMEM((_CHUNK,), jnp.int32)] * 3,
          [pltpu.VMEM((_GCH,), jnp.int32)] * _NG,
          pltpu.VMEM((_CHUNK, d_out), jnp.float32),
          pltpu.SemaphoreType.DMA,
      ],
  )
  def lookup(te_hbm, tab_hbm, out_hbm, f_v, idx_v, rows_v, gsem):
    nc = mesh.num_cores
    wid = lax.axis_index("s") * nc + lax.axis_index("c")
    base = wid * rows_per_worker

    def chunk_body(ch, carry):
      row0 = base + ch * _CHUNK
      for k in range(3):
        pltpu.sync_copy(te_hbm.at[pl.ds(k * n_rows + row0, _CHUNK)], f_v[k])
      copies = []
      for g in range(_NG):
        def idx_body(jj, carry2, g=g):
          p = g * _GCH + jj * 16
          f2 = f_v[0][pl.ds(p, 16)]
          f3 = f_v[1][pl.ds(p, 16)]
          f4 = f_v[2][pl.ds(p, 16)]
          day = lax.div(lax.rem(f3, 24) * 60 + lax.rem(f4, 60), 5)
          idx_v[g][pl.ds(jj * 16, 16)] = lax.rem(f2, 7) * 288 + day
          return carry2

        lax.fori_loop(0, _GCH // 16, idx_body, 0)
        copies.append(
            pltpu.async_copy(
                tab_hbm.at[idx_v[g]],
                rows_v.at[pl.ds(g * _GCH, _GCH)],
                gsem,
            )
        )
      for c in copies:
        c.wait()
      pltpu.sync_copy(rows_v, out_hbm.at[pl.ds(row0, _CHUNK)])
      return carry

    lax.fori_loop(0, n_chunks, chunk_body, 0)

  return lookup


def kernel(TE, day_table, week_table):
  Bv, Tv, _ = TE.shape
  n_rows = Bv * Tv
  d = day_table.shape[1]
  fused = jnp.concatenate(
      [jnp.tile(day_table, (7, 1)), jnp.repeat(week_table, 288, axis=0)],
      axis=1,
  )
  te_fields = (
      TE.astype(jnp.int32).reshape(n_rows, 5)[:, 2:5].T.reshape(3 * n_rows)
  )
  out = _make_lookup(n_rows, 2 * d, fused.shape[0])(te_fields, fused)
  return out.reshape(Bv, Tv, 2 * d)


# speedup vs baseline: 6.4272x; 1.0999x over previous
import functools

import jax
import jax.numpy as jnp
from jax import lax
from jax.experimental import pallas as pl
from jax.experimental.pallas import tpu as pltpu
from jax.experimental.pallas import tpu_sc as plsc

_NUM_WORKERS = 32
_GCH = 128
_NB = 5


def _make_lookup(n_rows: int, d_out: int):
  assert n_rows % (_NUM_WORKERS * _GCH * _NB) == 0
  rows_per_worker = n_rows // _NUM_WORKERS
  n_groups = rows_per_worker // (_GCH * _NB)
  mesh = plsc.VectorSubcoreMesh(core_axis_name="c", subcore_axis_name="s")

  @functools.partial(
      pl.kernel,
      out_type=jax.ShapeDtypeStruct((n_rows, d_out), jnp.float32),
      mesh=mesh,
      scratch_types=[
          [pltpu.VMEM((rows_per_worker,), jnp.int32)] * 3,
          [pltpu.VMEM((_GCH,), jnp.int32)] * _NB,
          [pltpu.VMEM((_GCH, d_out), jnp.float32)] * _NB,
          pltpu.SemaphoreType.DMA,
          pltpu.SemaphoreType.DMA,
      ],
  )
  def lookup(te_hbm, tab_hbm, out_hbm, f_v, idx_v, rows_v, gsem, osem):
    nc = mesh.num_cores
    wid = lax.axis_index("s") * nc + lax.axis_index("c")
    base = wid * rows_per_worker

    for k in range(3):
      pltpu.sync_copy(te_hbm.at[pl.ds(k * n_rows + base, rows_per_worker)],
                      f_v[k])

    def wait_gather(b):
      pltpu.make_async_copy(tab_hbm.at[idx_v[b]], rows_v[b], gsem).wait()

    def fire_out(b, blk):
      pltpu.async_copy(rows_v[b], out_hbm.at[pl.ds(base + blk * _GCH, _GCH)],
                       osem)

    def wait_out(b):
      pltpu.make_async_copy(rows_v[b], out_hbm.at[pl.ds(0, _GCH)],
                            osem).wait()

    def group_body(j2, carry):
      for b in range(_NB):
        blk = j2 * _NB + b
        @pl.when(j2 > 0)
        def _(b=b):
          wait_out(b)

        def idx_body(jj, carry2, b=b, blk=blk):
          p = blk * _GCH + jj * 16
          f2 = f_v[0][pl.ds(p, 16)]
          f3 = f_v[1][pl.ds(p, 16)]
          f4 = f_v[2][pl.ds(p, 16)]
          day = lax.div(lax.rem(f3, 24) * 60 + lax.rem(f4, 60), 5)
          idx_v[b][pl.ds(jj * 16, 16)] = lax.rem(f2, 7) * 288 + day
          return carry2

        lax.fori_loop(0, _GCH // 16, idx_body, 0)
        pltpu.async_copy(tab_hbm.at[idx_v[b]], rows_v[b], gsem)

        if b > 0:
          wait_gather(b - 1)
          fire_out(b - 1, blk - 1)
        else:
          @pl.when(j2 > 0)
          def _(blk=blk):
            wait_gather(_NB - 1)
            fire_out(_NB - 1, blk - 1)
      return carry

    lax.fori_loop(0, n_groups, group_body, 0)

    wait_gather(_NB - 1)
    fire_out(_NB - 1, n_groups * _NB - 1)
    for b in range(_NB):
      wait_out(b)

  return lookup


def kernel(TE, day_table, week_table):
  Bv, Tv, _ = TE.shape
  n_rows = Bv * Tv
  d = day_table.shape[1]
  fused = jnp.concatenate(
      [jnp.tile(day_table, (7, 1)), jnp.repeat(week_table, 288, axis=0)],
      axis=1,
  )
  te_fields = (
      TE.astype(jnp.int32).reshape(n_rows, 5)[:, 2:5].T.reshape(3 * n_rows)
  )
  out = _make_lookup(n_rows, 2 * d)(te_fields, fused)
  return out.reshape(Bv, Tv, 2 * d)
